# baseline (speedup 1.0000x reference)
import functools

import jax
import jax.numpy as jnp
from jax import lax
from jax.experimental import pallas as pl
from jax.experimental.pallas import tpu as pltpu
from jax.experimental.pallas import tpu_sc as plsc

NUM_CORES = 2
NUM_SUBCORES = 16
LANES = 16
NUM_WORKERS = NUM_CORES * NUM_SUBCORES

LATENT_DIM = 32
CHUNK = 128
NUM_CHUNKS = 4
ROWS_PER_WORKER = CHUNK * NUM_CHUNKS
GROUPS = ROWS_PER_WORKER // LANES


def _gmf_body(emb_u_hbm, emb_i_hbm, uidx_hbm, iidx_hbm, out_hbm,
              uidx_v, iidx_v, u_rows, i_rows, out_v, sem):
  wid = lax.axis_index("s") * NUM_CORES + lax.axis_index("c")

  pltpu.sync_copy(uidx_hbm.at[wid], uidx_v)
  pltpu.sync_copy(iidx_hbm.at[wid], iidx_v)

  copies = []
  for j in range(NUM_CHUNKS):
    rows = pl.ds(j * CHUNK, CHUNK)
    copies.append(pltpu.async_copy(emb_u_hbm.at[uidx_v.at[j]], u_rows.at[rows], sem))
    copies.append(pltpu.async_copy(emb_i_hbm.at[iidx_v.at[j]], i_rows.at[rows], sem))
  for c in copies:
    c.wait()

  lane = lax.iota(jnp.int32, LANES)

  def group(g, carry):
    row = g * LANES + lane
    acc = jnp.zeros((LANES,), jnp.float32)
    for d in range(LATENT_DIM):
      col = jnp.full((LANES,), d, jnp.int32)
      uv = plsc.load_gather(u_rows, [row, col])
      iv = plsc.load_gather(i_rows, [row, col])
      acc = acc + uv * iv
    rating = 1.0 / (1.0 + jnp.exp(-acc))
    out_v[pl.ds(g * LANES, LANES)] = rating
    return carry

  lax.fori_loop(0, GROUPS, group, 0, unroll=False)

  pltpu.sync_copy(out_v, out_hbm.at[wid])


@functools.partial(jax.jit, static_argnums=())
def _gmf(user_idx, item_idx, emb_user, emb_item):
  mesh = plsc.VectorSubcoreMesh(
      core_axis_name="c", subcore_axis_name="s",
      num_cores=NUM_CORES, num_subcores=NUM_SUBCORES)
  run = pl.kernel(
      _gmf_body,
      out_type=jax.ShapeDtypeStruct((NUM_WORKERS, ROWS_PER_WORKER), jnp.float32),
      mesh=mesh,
      compiler_params=pltpu.CompilerParams(
          needs_layout_passes=False, use_tc_tiling_on_sc=False),
      scratch_types=[
          pltpu.VMEM((NUM_CHUNKS, CHUNK), jnp.int32),
          pltpu.VMEM((NUM_CHUNKS, CHUNK), jnp.int32),
          pltpu.VMEM((ROWS_PER_WORKER, LATENT_DIM), jnp.float32),
          pltpu.VMEM((ROWS_PER_WORKER, LATENT_DIM), jnp.float32),
          pltpu.VMEM((ROWS_PER_WORKER,), jnp.float32),
          pltpu.SemaphoreType.DMA,
      ],
  )
  return run(emb_user, emb_item, user_idx, item_idx)


def kernel(user_indices, item_indices, emb_user, emb_item):
  batch = user_indices.shape[0]
  uidx = user_indices.astype(jnp.int32).reshape(NUM_WORKERS, NUM_CHUNKS, CHUNK)
  iidx = item_indices.astype(jnp.int32).reshape(NUM_WORKERS, NUM_CHUNKS, CHUNK)
  out = _gmf(uidx, iidx, emb_user, emb_item)
  return out.reshape(batch)
